# baseline (speedup 1.0000x reference)
import jax
import jax.numpy as jnp
from jax import lax
from jax.experimental import pallas as pl
from jax.experimental.pallas import tpu as pltpu
from jax.experimental.pallas import tpu_sc as plsc

VOCAB_PAD = 1024

NC = 2
NS = 16
NW = NC * NS

R = 200
CT = 16384
CPW = CT // NW
RBLK = 8
NCHUNK = R // RBLK
NBUF = 4
NTRIP = (NCHUNK - 1) // NBUF
L = 16
NSLICE = CPW // L


def _body(probs_hbm, alias_hbm, index_hbm, prob_hbm, out_hbm,
          probs_v, alias_v, idx_v0, idx_v1, idx_v2, idx_v3, prob_v0, prob_v1,
          prob_v2, prob_v3, out_v0, out_v1, out_v2, out_v3,
          si0, si1, si2, si3, sp0, sp1, sp2, sp3, so0, so1, so2, so3):
    wid = lax.axis_index("s") * NC + lax.axis_index("c")
    c0 = wid * CPW

    pltpu.sync_copy(probs_hbm, probs_v)
    pltpu.sync_copy(alias_hbm, alias_v)

    si = (si0, si1, si2, si3)
    sp = (sp0, sp1, sp2, sp3)
    so = (so0, so1, so2, so3)
    idx_b = (idx_v0, idx_v1, idx_v2, idx_v3)
    prob_b = (prob_v0, prob_v1, prob_v2, prob_v3)
    out_b = (out_v0, out_v1, out_v2, out_v3)

    def issue_in(j, b):
        rn = j * RBLK
        pltpu.async_copy(index_hbm.at[pl.ds(rn, RBLK), pl.ds(c0, CPW)],
                         idx_b[b], si[b])
        pltpu.async_copy(prob_hbm.at[pl.ds(rn, RBLK), pl.ds(c0, CPW)],
                         prob_b[b], sp[b])

    def wait_in(b):
        pltpu.make_async_copy(index_hbm.at[pl.ds(0, RBLK), pl.ds(c0, CPW)],
                              idx_b[b], si[b]).wait()
        pltpu.make_async_copy(prob_hbm.at[pl.ds(0, RBLK), pl.ds(c0, CPW)],
                              prob_b[b], sp[b]).wait()

    def issue_out(j, b):
        pltpu.async_copy(out_b[b],
                         out_hbm.at[pl.ds(j * RBLK, RBLK), pl.ds(c0, CPW)],
                         so[b])

    def wait_out(b):
        pltpu.make_async_copy(out_b[b],
                              out_hbm.at[pl.ds(0, RBLK), pl.ds(c0, CPW)],
                              so[b]).wait()

    def compute(b):
        ib = idx_b[b]
        pb = prob_b[b]
        ob = out_b[b]

        @plsc.parallel_loop(0, NSLICE, 1, unroll=1)
        def _(c):
            s = pl.ds(c * L, L)
            for r in range(RBLK):
                idx = ib[r, s]
                pv = pb[r, s]
                pa = plsc.load_gather(probs_v, [idx])
                al = plsc.load_gather(alias_v, [idx])
                ob[r, s] = jnp.where(pv < pa, idx, al)

    for b in range(NBUF):
        issue_in(b, b)

    def ring(k, _):
        j0 = k * NBUF
        for b in range(NBUF):
            j = j0 + b
            wait_in(b)

            @pl.when(k > 0)
            def _():
                wait_out(b)

            compute(b)
            issue_out(j, b)

            @pl.when(j + NBUF < NCHUNK)
            def _():
                issue_in(j + NBUF, b)
        return 0

    lax.fori_loop(0, NTRIP, ring, 0)

    wait_in(0)
    wait_out(0)
    compute(0)
    issue_out(NCHUNK - 1, 0)

    wait_out(1)
    wait_out(2)
    wait_out(3)
    wait_out(0)


@jax.jit
def _sample(probs_pad, alias_pad, index_t, prob_t):
    mesh = plsc.VectorSubcoreMesh(core_axis_name="c", subcore_axis_name="s")
    return pl.kernel(
        _body,
        out_type=jax.ShapeDtypeStruct((R, CT), jnp.int32),
        mesh=mesh,
        scratch_types=[
            pltpu.VMEM((VOCAB_PAD,), jnp.float32),
            pltpu.VMEM((VOCAB_PAD,), jnp.int32),
        ] + [pltpu.VMEM((RBLK, CPW), jnp.int32)] * 4
          + [pltpu.VMEM((RBLK, CPW), jnp.float32)] * 4
          + [pltpu.VMEM((RBLK, CPW), jnp.int32)] * 4
          + [pltpu.SemaphoreType.DMA] * 12,
        compiler_params=pltpu.CompilerParams(needs_layout_passes=False),
    )(probs_pad, alias_pad, index_t, prob_t)


def kernel(probs, alias, index, prob):
    v = probs.shape[0]
    probs_pad = jnp.pad(probs, (0, VOCAB_PAD - v))
    alias_pad = jnp.pad(alias, (0, VOCAB_PAD - v))
    out_t = _sample(probs_pad, alias_pad,
                    jnp.swapaxes(index, 0, 1), jnp.swapaxes(prob, 0, 1))
    return jnp.swapaxes(out_t, 0, 1)


# speedup vs baseline: 1.0571x; 1.0571x over previous
import jax
import jax.numpy as jnp
from jax import lax
from jax.experimental import pallas as pl
from jax.experimental.pallas import tpu as pltpu
from jax.experimental.pallas import tpu_sc as plsc

VOCAB_PAD = 1024

NC = 2
NS = 16
NW = NC * NS

R = 200
CT = 16384
CPW = CT // NW
RBLK = 8
NCHUNK = R // RBLK
NBUF = 3
NTRIP = (NCHUNK - 1) // NBUF
L = 16
NSLICE = CPW // L


def _body(probs_hbm, alias_hbm, index_hbm, prob_hbm, out_hbm,
          probs_v, alias_v, idx_v0, idx_v1, idx_v2, prob_v0, prob_v1,
          prob_v2, out_v0, out_v1, out_v2,
          si0, si1, si2, sp0, sp1, sp2, so0, so1, so2, st0, st1):
    wid = lax.axis_index("s") * NC + lax.axis_index("c")
    c0 = wid * CPW

    tdesc = (pltpu.async_copy(probs_hbm, probs_v, st0),
             pltpu.async_copy(alias_hbm, alias_v, st1))

    si = (si0, si1, si2)
    sp = (sp0, sp1, sp2)
    so = (so0, so1, so2)
    idx_b = (idx_v0, idx_v1, idx_v2)
    prob_b = (prob_v0, prob_v1, prob_v2)
    out_b = (out_v0, out_v1, out_v2)

    def issue_in(j, b):
        rn = j * RBLK
        pltpu.async_copy(index_hbm.at[pl.ds(rn, RBLK), pl.ds(c0, CPW)],
                         idx_b[b], si[b])
        pltpu.async_copy(prob_hbm.at[pl.ds(rn, RBLK), pl.ds(c0, CPW)],
                         prob_b[b], sp[b])

    def wait_in(b):
        pltpu.make_async_copy(index_hbm.at[pl.ds(0, RBLK), pl.ds(c0, CPW)],
                              idx_b[b], si[b]).wait()
        pltpu.make_async_copy(prob_hbm.at[pl.ds(0, RBLK), pl.ds(c0, CPW)],
                              prob_b[b], sp[b]).wait()

    def issue_out(j, b):
        pltpu.async_copy(out_b[b],
                         out_hbm.at[pl.ds(j * RBLK, RBLK), pl.ds(c0, CPW)],
                         so[b])

    def wait_out(b):
        pltpu.make_async_copy(out_b[b],
                              out_hbm.at[pl.ds(0, RBLK), pl.ds(c0, CPW)],
                              so[b]).wait()

    def compute(b):
        ib = idx_b[b]
        pb = prob_b[b]
        ob = out_b[b]

        @plsc.parallel_loop(0, NSLICE, 1, unroll=1)
        def _(c):
            s = pl.ds(c * L, L)
            for r in range(RBLK):
                idx = ib[r, s]
                pv = pb[r, s]
                pa = plsc.load_gather(probs_v, [idx])
                al = plsc.load_gather(alias_v, [idx])
                ob[r, s] = jnp.where(pv < pa, idx, al)

    for b in range(NBUF):
        issue_in(b, b)
    tdesc[0].wait()
    tdesc[1].wait()

    def ring(k, _):
        j0 = k * NBUF
        for b in range(NBUF):
            j = j0 + b
            wait_in(b)

            @pl.when(k > 0)
            def _():
                wait_out(b)

            compute(b)
            issue_out(j, b)

            @pl.when(j + NBUF < NCHUNK)
            def _():
                issue_in(j + NBUF, b)
        return 0

    lax.fori_loop(0, NTRIP, ring, 0)

    wait_in(0)
    wait_out(0)
    compute(0)
    issue_out(NCHUNK - 1, 0)

    wait_out(1)
    wait_out(2)
    wait_out(0)


@jax.jit
def _sample(probs_pad, alias_pad, index_t, prob_t):
    mesh = plsc.VectorSubcoreMesh(core_axis_name="c", subcore_axis_name="s")
    return pl.kernel(
        _body,
        out_type=jax.ShapeDtypeStruct((R, CT), jnp.int32),
        mesh=mesh,
        scratch_types=[
            pltpu.VMEM((VOCAB_PAD,), jnp.float32),
            pltpu.VMEM((VOCAB_PAD,), jnp.int32),
        ] + [pltpu.VMEM((RBLK, CPW), jnp.int32)] * 3
          + [pltpu.VMEM((RBLK, CPW), jnp.float32)] * 3
          + [pltpu.VMEM((RBLK, CPW), jnp.int32)] * 3
          + [pltpu.SemaphoreType.DMA] * 11,
        compiler_params=pltpu.CompilerParams(needs_layout_passes=False),
    )(probs_pad, alias_pad, index_t, prob_t)


def kernel(probs, alias, index, prob):
    v = probs.shape[0]
    probs_pad = jnp.pad(probs, (0, VOCAB_PAD - v))
    alias_pad = jnp.pad(alias, (0, VOCAB_PAD - v))
    out_t = _sample(probs_pad, alias_pad,
                    jnp.swapaxes(index, 0, 1), jnp.swapaxes(prob, 0, 1))
    return jnp.swapaxes(out_t, 0, 1)
